# baseline (speedup 1.0000x reference)
import functools

import jax
import jax.numpy as jnp
from jax import lax
from jax.experimental import pallas as pl
from jax.experimental.pallas import tpu as pltpu
from jax.experimental.pallas import tpu_sc as plsc

_N = 32
_A = 16
_XD = 512
_L = 16
_NW = 32


def _head_body(x_ref, pmf_ref, wrow_ref, w1_ref, w2_ref, b_ref, avail_ref,
               act_ref, alp_ref):
    x = x_ref[...]
    pmfs = pmf_ref[...] * wrow_ref[...]
    logits = jnp.dot(x, w1_ref[...], preferred_element_type=jnp.float32)
    logits = logits + jnp.dot(pmfs, w2_ref[...], preferred_element_type=jnp.float32)
    logits = logits + b_ref[...]
    masked = jnp.where(avail_ref[...] > 0, logits, -1e10)
    m = jnp.max(masked, axis=-1, keepdims=True)
    lse = jnp.log(jnp.sum(jnp.exp(masked - m), axis=-1, keepdims=True))
    act_ref[...] = jnp.argmax(masked, axis=-1, keepdims=True).astype(jnp.int32)
    alp_ref[...] = -lse


def _sc_father(pmf_hbm, wv_hbm, father_hbm,
               fbuf0, fbuf1, prow0, prow1, wv, sem0, sem1):
    nn = _N * _N
    row_w = nn * _A
    wid = lax.axis_index("s") * 2 + lax.axis_index("c")
    rows = 1024 // _NW
    base = wid * rows

    pltpu.sync_copy(wv_hbm, wv)

    zeros = jnp.zeros((_L,), jnp.float32)

    def zero_body(i, c):
        fbuf0[pl.ds(i * _L, _L)] = zeros
        fbuf1[pl.ds(i * _L, _L)] = zeros
        return c

    lax.fori_loop(0, row_w // _L, zero_body, 0)

    lane_off = lax.iota(jnp.int32, _L) * _A

    def do_row(r, fbuf, prow):
        pltpu.sync_copy(pmf_hbm.at[r], prow)

        def g_body(g, _):
            off = g * _L
            vals = prow[pl.ds(off, _L)] * wv[pl.ds(off, _L)]
            idx = lane_off + g * (_L * _A)
            plsc.store_scatter(fbuf, [idx], vals)
            return 0

        lax.fori_loop(0, nn // _L, g_body, 0, unroll=8)

    def start(fbuf, r, sem):
        pltpu.make_async_copy(fbuf, father_hbm.at[r], sem).start()

    def wait(fbuf, r, sem):
        pltpu.make_async_copy(fbuf, father_hbm.at[r], sem).wait()

    do_row(base + 0, fbuf0, prow0)
    start(fbuf0, base + 0, sem0)
    do_row(base + 1, fbuf1, prow1)
    start(fbuf1, base + 1, sem1)

    def step_body(s, _):
        r0 = base + s * 2
        wait(fbuf0, r0, sem0)
        do_row(r0, fbuf0, prow0)
        start(fbuf0, r0, sem0)
        r1 = r0 + 1
        wait(fbuf1, r1, sem1)
        do_row(r1, fbuf1, prow1)
        start(fbuf1, r1, sem1)
        return 0

    lax.fori_loop(1, rows // 2, step_body, 0)
    wait(fbuf0, base, sem0)
    wait(fbuf1, base, sem1)


def kernel(x, parents_mask, available_actions, father_action_weights, W, b,
           deterministic=True):
    T = x.shape[0]
    n = _N
    A = _A
    nn = n * n
    pmf2d = parents_mask.reshape(T, nn).astype(jnp.float32)
    wvec = jnp.tile(father_action_weights, n)
    W1 = W[:_XD]
    W2 = W[_XD::A]

    row_w = nn * A
    sc_father = pl.kernel(
        _sc_father,
        out_type=jax.ShapeDtypeStruct((T, row_w), jnp.float32),
        mesh=plsc.VectorSubcoreMesh(core_axis_name="c", subcore_axis_name="s"),
        compiler_params=pltpu.CompilerParams(needs_layout_passes=False),
        scratch_types=[
            pltpu.VMEM((row_w,), jnp.float32),
            pltpu.VMEM((row_w,), jnp.float32),
            pltpu.VMEM((nn,), jnp.float32),
            pltpu.VMEM((nn,), jnp.float32),
            pltpu.VMEM((nn,), jnp.float32),
            pltpu.SemaphoreType.DMA,
            pltpu.SemaphoreType.DMA,
        ],
    )
    father = sc_father(pmf2d, wvec)

    actions, alp = pl.pallas_call(
        _head_body,
        in_specs=[pl.BlockSpec(memory_space=pltpu.VMEM)] * 7,
        out_specs=[pl.BlockSpec(memory_space=pltpu.VMEM)] * 2,
        out_shape=[
            jax.ShapeDtypeStruct((T, 1), jnp.int32),
            jax.ShapeDtypeStruct((T, 1), jnp.float32),
        ],
    )(x, pmf2d, wvec.reshape(1, nn), W1, W2, b.reshape(1, A),
      available_actions)

    return (actions, alp, father)


# speedup vs baseline: 1.3549x; 1.3549x over previous
import jax
import jax.numpy as jnp
from jax import lax
from jax.experimental import pallas as pl
from jax.experimental.pallas import tpu as pltpu
from jax.experimental.pallas import tpu_sc as plsc

_N = 32
_A = 16
_XD = 512
_L = 16
_NW = 32


def _head_body(x_ref, pmf_ref, wrow_ref, w1_ref, w2_ref, b_ref, avail_ref,
               act_ref, alp_ref):
    x = x_ref[...]
    pmfs = pmf_ref[...].astype(jnp.float32) * wrow_ref[...]
    logits = jnp.dot(x, w1_ref[...], preferred_element_type=jnp.float32)
    logits = logits + jnp.dot(pmfs, w2_ref[...], preferred_element_type=jnp.float32)
    logits = logits + b_ref[...]
    masked = jnp.where(avail_ref[...] > 0, logits, -1e10)
    m = jnp.max(masked, axis=-1, keepdims=True)
    lse = jnp.log(jnp.sum(jnp.exp(masked - m), axis=-1, keepdims=True))
    act_ref[...] = jnp.argmax(masked, axis=-1, keepdims=True).astype(jnp.int32)
    alp_ref[...] = -lse


def _sc_father(pmf_hbm, w_hbm, father_hbm,
               fbuf0, fbuf1, pbuf, wv, sem0, sem1):
    nn = _N * _N
    row_w = nn * _A
    wid = lax.axis_index("s") * 2 + lax.axis_index("c")
    rows = 1024 // _NW
    base = wid * rows

    pltpu.sync_copy(w_hbm, wv)
    pltpu.sync_copy(pmf_hbm.at[pl.ds(base, rows)], pbuf)

    zeros = jnp.zeros((_L,), jnp.float32)

    def zero_body(i, c):
        fbuf0[pl.ds(i * _L, _L)] = zeros
        fbuf1[pl.ds(i * _L, _L)] = zeros
        return c

    lax.fori_loop(0, row_w // _L, zero_body, 0)

    lane_off = lax.iota(jnp.int32, _L) * _A
    w_lo = wv[pl.ds(0, _L)]
    w_hi = wv[pl.ds(_L, _L)]

    def do_row(rr, fbuf):
        def g_body(h, _):
            g0 = h * 2
            pv0 = pbuf[rr, pl.ds(g0 * _L, _L)].astype(jnp.float32)
            plsc.store_scatter(fbuf, [lane_off + g0 * (_L * _A)], pv0 * w_lo)
            g1 = g0 + 1
            pv1 = pbuf[rr, pl.ds(g1 * _L, _L)].astype(jnp.float32)
            plsc.store_scatter(fbuf, [lane_off + g1 * (_L * _A)], pv1 * w_hi)
            return 0

        lax.fori_loop(0, nn // (2 * _L), g_body, 0, unroll=4)

    def start(fbuf, r, sem):
        pltpu.make_async_copy(fbuf, father_hbm.at[r], sem).start()

    def wait(fbuf, r, sem):
        pltpu.make_async_copy(fbuf, father_hbm.at[r], sem).wait()

    do_row(0, fbuf0)
    start(fbuf0, base + 0, sem0)
    do_row(1, fbuf1)
    start(fbuf1, base + 1, sem1)

    def step_body(s, _):
        rr0 = s * 2
        wait(fbuf0, base + rr0, sem0)
        do_row(rr0, fbuf0)
        start(fbuf0, base + rr0, sem0)
        rr1 = rr0 + 1
        wait(fbuf1, base + rr1, sem1)
        do_row(rr1, fbuf1)
        start(fbuf1, base + rr1, sem1)
        return 0

    lax.fori_loop(1, rows // 2, step_body, 0)
    wait(fbuf0, base, sem0)
    wait(fbuf1, base, sem1)


def kernel(x, parents_mask, available_actions, father_action_weights, W, b,
           deterministic=True):
    T = x.shape[0]
    n = _N
    A = _A
    nn = n * n
    pmf2d = parents_mask.reshape(T, nn)
    wvec = jnp.tile(father_action_weights, n)
    W1 = W[:_XD]
    W2 = W[_XD::A]

    row_w = nn * A
    rows = T // _NW
    sc_father = pl.kernel(
        _sc_father,
        out_type=jax.ShapeDtypeStruct((T, row_w), jnp.float32),
        mesh=plsc.VectorSubcoreMesh(core_axis_name="c", subcore_axis_name="s"),
        compiler_params=pltpu.CompilerParams(needs_layout_passes=False),
        scratch_types=[
            pltpu.VMEM((row_w,), jnp.float32),
            pltpu.VMEM((row_w,), jnp.float32),
            pltpu.VMEM((rows, nn), jnp.int32),
            pltpu.VMEM((_N,), jnp.float32),
            pltpu.SemaphoreType.DMA,
            pltpu.SemaphoreType.DMA,
        ],
    )
    father = sc_father(pmf2d, father_action_weights)

    actions, alp = pl.pallas_call(
        _head_body,
        in_specs=[pl.BlockSpec(memory_space=pltpu.VMEM)] * 7,
        out_specs=[pl.BlockSpec(memory_space=pltpu.VMEM)] * 2,
        out_shape=[
            jax.ShapeDtypeStruct((T, 1), jnp.int32),
            jax.ShapeDtypeStruct((T, 1), jnp.float32),
        ],
    )(x, pmf2d, wvec.reshape(1, nn), W1, W2, b.reshape(1, A),
      available_actions)

    return (actions, alp, father)
